# baseline (speedup 1.0000x reference)
import jax
import jax.numpy as jnp
from jax import lax
from jax.experimental import pallas as pl
from jax.experimental.pallas import tpu as pltpu
from jax.experimental.pallas import tpu_sc as plsc

N_NODES = 10000
N_EDGES = 320000
D = 128

NC, NS = 2, 16
NW = NC * NS
E_PER_W = N_EDGES // NW
CHUNK = 64
NHALF = 3
CPH = -(-(-(-E_PER_W // CHUNK)) // NHALF)
NCH = NHALF * CPH
E_PAD_W = NCH * CHUNK
ACC_ROWS = N_NODES + 8
NB = 4

NZF = ACC_ROWS // CHUNK
ZTAIL = ACC_ROWS - NZF * CHUNK
NWF = N_NODES // CHUNK
WTAIL = N_NODES - NWF * CHUNK


def _sc_aggregate_body(idx_hbm, x_hbm, out_hbm,
                       idx_i, rows, idxg, idxs, acc, gsems, ssems):
    c = lax.axis_index("c")
    s = lax.axis_index("s")
    wid = c * NS + s

    idx_pref = pltpu.async_copy(idx_hbm.at[wid, pl.ds(0, CPH)], idx_i,
                                gsems[0])

    zero = jnp.zeros((16,), jnp.float32)

    def zbuf_body(i, carry):
        rows[0][i // 8, pl.ds((i % 8) * 16, 16)] = zero
        return carry

    lax.fori_loop(0, CHUNK * (D // 16), zbuf_body, 0)

    n_z = (NZF - s + NS - 1) // NS

    def zacc_body(t, carry):
        j = s + t * NS
        pltpu.sync_copy(rows[0], acc.at[pl.ds(j * CHUNK, CHUNK)])
        return carry

    lax.fori_loop(0, n_z, zacc_body, 0)

    if ZTAIL:
        @pl.when(s == NS - 1)
        def _():
            pltpu.sync_copy(rows[0].at[pl.ds(0, ZTAIL)],
                            acc.at[pl.ds(NZF * CHUNK, ZTAIL)])

    plsc.subcore_barrier()
    idx_pref.wait()

    def stage_idx(j, b):
        for k in range(CHUNK // 16):
            idxg[b][pl.ds(k * 16, 16)] = idx_i[j, 0, pl.ds(k * 16, 16)]
            idxs[b][pl.ds(k * 16, 16)] = idx_i[j, 1, pl.ds(k * 16, 16)]

    def fire_gather(b):
        pltpu.async_copy(x_hbm.at[idxg[b]], rows[b], gsems[b])

    def wait_gather(b):
        pltpu.make_async_copy(x_hbm.at[idxg[b]], rows[b],
                              gsems[b]).wait()

    def fire_scatter(b):
        pltpu.async_copy(rows[b], acc.at[idxs[b]], ssems[b], add=True)

    def wait_scatter(b):
        pltpu.make_async_copy(rows[b], acc.at[idxs[b]],
                              ssems[b]).wait()

    def pipe_body(t, carry):
        j0 = t * NB
        for b in range(NB):
            @pl.when(j0 + b < CPH)
            def _():
                wait_gather(b)
                fire_scatter(b)
        for b in range(NB):
            @pl.when(j0 + b + NB < CPH)
            def _():
                wait_scatter(b)
                stage_idx(j0 + b + NB, b)
                fire_gather(b)
        return carry

    for h in range(NHALF):
        if h > 0:
            pltpu.sync_copy(idx_hbm.at[wid, pl.ds(h * CPH, CPH)], idx_i)
        for b in range(NB):
            stage_idx(b, b)
            fire_gather(b)
        lax.fori_loop(0, -(-CPH // NB), pipe_body, 0)
        for b in range(NB):
            wait_scatter(b)
    plsc.subcore_barrier()

    n_wb = (NWF - s + NS - 1) // NS

    def wb_body(t, carry):
        j = s + t * NS
        pltpu.sync_copy(acc.at[pl.ds(j * CHUNK, CHUNK)],
                        out_hbm.at[c, pl.ds(j * CHUNK, CHUNK)])
        return carry

    lax.fori_loop(0, n_wb, wb_body, 0)

    if WTAIL:
        @pl.when(s == NS - 1)
        def _():
            pltpu.sync_copy(acc.at[pl.ds(NWF * CHUNK, WTAIL)],
                            out_hbm.at[c, pl.ds(NWF * CHUNK, WTAIL)])


def _sc_aggregate(edge_index, x):
    src = edge_index[0].reshape(NW, E_PER_W)
    dst = edge_index[1].reshape(NW, E_PER_W)
    pad = E_PAD_W - E_PER_W
    src3 = jnp.pad(src, ((0, 0), (0, pad))).reshape(NW, NCH, CHUNK)
    dst3 = jnp.pad(dst, ((0, 0), (0, pad)),
                   constant_values=N_NODES).reshape(NW, NCH, CHUNK)
    idx4 = jnp.stack([src3, dst3], axis=2)

    mesh = plsc.VectorSubcoreMesh(core_axis_name="c", subcore_axis_name="s")
    k = pl.kernel(
        _sc_aggregate_body,
        out_type=jax.ShapeDtypeStruct((NC, N_NODES, D), jnp.float32),
        mesh=mesh,
        scratch_types=[
            pltpu.VMEM((CPH, 2, CHUNK), jnp.int32),
            [pltpu.VMEM((CHUNK, D), jnp.float32) for _ in range(NB)],
            [pltpu.VMEM((CHUNK,), jnp.int32) for _ in range(NB)],
            [pltpu.VMEM((CHUNK,), jnp.int32) for _ in range(NB)],
            pltpu.VMEM_SHARED((ACC_ROWS, D), jnp.float32),
            [pltpu.SemaphoreType.DMA for _ in range(NB)],
            [pltpu.SemaphoreType.DMA for _ in range(NB)],
        ],
    )
    return k(idx4, x)


def _tc_head_body(x_ref, p_ref, ws_ref, wn_ref, b_ref, wo_ref, out_ref):
    agg = p_ref[0] + p_ref[1]
    h = (jnp.dot(x_ref[...], ws_ref[...], preferred_element_type=jnp.float32)
         + jnp.dot(agg, wn_ref[...], preferred_element_type=jnp.float32)
         + b_ref[...][None, :])
    h = jnp.maximum(h, 0.0)
    out_ref[...] = jnp.sum(h * wo_ref[...][None, :], axis=1)


def _tc_head(x, partials, W_self, W_nbr, b, w_out):
    return pl.pallas_call(
        _tc_head_body,
        out_shape=jax.ShapeDtypeStruct((N_NODES,), jnp.float32),
    )(x, partials, W_self, W_nbr, b, w_out)


def kernel(x, edge_index, W_self, W_nbr, b, w_out):
    partials = _sc_aggregate(edge_index, x)
    return _tc_head(x, partials, W_self, W_nbr, b, w_out)


# speedup vs baseline: 2.1761x; 2.1761x over previous
import jax
import jax.numpy as jnp
from jax import lax
from jax.experimental import pallas as pl
from jax.experimental.pallas import tpu as pltpu
from jax.experimental.pallas import tpu_sc as plsc

N_NODES = 10000
N_EDGES = 320000
D = 128

NC, NS = 2, 16
NW = NC * NS
E_PER_W = N_EDGES // NW
CHUNK = 80
NCH = E_PER_W // CHUNK
NHALF = 5
CPH = NCH // NHALF
NB = 3

NZC = N_NODES // CHUNK


def _sc_aggregate_body(src_hbm, dst_hbm, x_hbm, out_hbm,
                       src_i, dst_i, rows, acc, gsems, ssems):
    c = lax.axis_index("c")
    s = lax.axis_index("s")
    wid = c * NS + s

    pref = [
        pltpu.async_copy(src_hbm.at[wid, 0], src_i, gsems[0]),
        pltpu.async_copy(dst_hbm.at[wid, 0], dst_i, gsems[0]),
    ]

    zero = jnp.zeros((16,), jnp.float32)

    def zbuf_body(i, carry):
        rows[0][i // 8, pl.ds((i % 8) * 16, 16)] = zero
        return carry

    lax.fori_loop(0, CHUNK * (D // 16), zbuf_body, 0)

    n_z = (NZC - s + NS - 1) // NS

    def zacc_body(t, carry):
        j = s + t * NS
        pltpu.sync_copy(rows[0], acc.at[pl.ds(j * CHUNK, CHUNK)])
        return carry

    lax.fori_loop(0, n_z, zacc_body, 0)
    plsc.subcore_barrier()
    for p in pref:
        p.wait()

    def fire_gather(j, b):
        pltpu.async_copy(x_hbm.at[src_i.at[j]], rows[b], gsems[b])

    def wait_gather(b):
        pltpu.make_async_copy(x_hbm.at[src_i.at[0]], rows[b],
                              gsems[b]).wait()

    def fire_scatter(j, b):
        pltpu.async_copy(rows[b], acc.at[dst_i.at[j]], ssems[b], add=True)

    def wait_scatter(b):
        pltpu.make_async_copy(rows[b], acc.at[dst_i.at[0]],
                              ssems[b]).wait()

    def pipe_body(t, carry):
        j0 = t * NB
        for b in range(NB):
            @pl.when(j0 + b < CPH)
            def _():
                wait_gather(b)
                fire_scatter(j0 + b, b)
        for b in range(NB):
            @pl.when(j0 + b + NB < CPH)
            def _():
                wait_scatter(b)
                fire_gather(j0 + b + NB, b)
        return carry

    for h in range(NHALF):
        if h > 0:
            pltpu.sync_copy(src_hbm.at[wid, h], src_i)
            pltpu.sync_copy(dst_hbm.at[wid, h], dst_i)
        for b in range(NB):
            fire_gather(b, b)
        lax.fori_loop(0, -(-CPH // NB), pipe_body, 0)
        for b in range(NB):
            wait_scatter(b)
    plsc.subcore_barrier()

    n_wb = (NZC - s + NS - 1) // NS

    def wb_body(t, carry):
        j = s + t * NS
        pltpu.sync_copy(acc.at[pl.ds(j * CHUNK, CHUNK)],
                        out_hbm.at[c, pl.ds(j * CHUNK, CHUNK)])
        return carry

    lax.fori_loop(0, n_wb, wb_body, 0)


def _sc_aggregate(edge_index, x):
    src3 = edge_index[0].reshape(NW, NHALF, CPH, CHUNK)
    dst3 = edge_index[1].reshape(NW, NHALF, CPH, CHUNK)

    mesh = plsc.VectorSubcoreMesh(core_axis_name="c", subcore_axis_name="s")
    k = pl.kernel(
        _sc_aggregate_body,
        out_type=jax.ShapeDtypeStruct((NC, N_NODES, D), jnp.float32),
        mesh=mesh,
        scratch_types=[
            pltpu.VMEM((CPH, CHUNK), jnp.int32),
            pltpu.VMEM((CPH, CHUNK), jnp.int32),
            [pltpu.VMEM((CHUNK, D), jnp.float32) for _ in range(NB)],
            pltpu.VMEM_SHARED((N_NODES, D), jnp.float32),
            [pltpu.SemaphoreType.DMA for _ in range(NB)],
            [pltpu.SemaphoreType.DMA for _ in range(NB)],
        ],
    )
    return k(src3, dst3, x)


def _tc_head_body(x_ref, p_ref, ws_ref, wn_ref, b_ref, wo_ref, out_ref):
    agg = p_ref[0] + p_ref[1]
    h = (jnp.dot(x_ref[...], ws_ref[...], preferred_element_type=jnp.float32)
         + jnp.dot(agg, wn_ref[...], preferred_element_type=jnp.float32)
         + b_ref[...][None, :])
    h = jnp.maximum(h, 0.0)
    out_ref[...] = jnp.sum(h * wo_ref[...][None, :], axis=1)


def _tc_head(x, partials, W_self, W_nbr, b, w_out):
    return pl.pallas_call(
        _tc_head_body,
        out_shape=jax.ShapeDtypeStruct((N_NODES,), jnp.float32),
    )(x, partials, W_self, W_nbr, b, w_out)


def kernel(x, edge_index, W_self, W_nbr, b, w_out):
    partials = _sc_aggregate(edge_index, x)
    return _tc_head(x, partials, W_self, W_nbr, b, w_out)


# speedup vs baseline: 2.3188x; 1.0656x over previous
import jax
import jax.numpy as jnp
from jax import lax
from jax.experimental import pallas as pl
from jax.experimental.pallas import tpu as pltpu
from jax.experimental.pallas import tpu_sc as plsc

N_NODES = 10000
N_EDGES = 320000
D = 128

NC, NS = 2, 16
NW = NC * NS
E_PER_W = N_EDGES // NW
CHUNK = 80
NCH = E_PER_W // CHUNK
NHALF = 5
CPH = NCH // NHALF
NB = 4

NZC = N_NODES // CHUNK


def _sc_aggregate_body(src_hbm, dst_hbm, x_hbm, out_hbm,
                       src_i, dst_i, rows, acc, gsems, ssems):
    c = lax.axis_index("c")
    s = lax.axis_index("s")
    wid = c * NS + s

    pref = [
        pltpu.async_copy(src_hbm.at[wid, 0], src_i, gsems[0]),
        pltpu.async_copy(dst_hbm.at[wid, 0], dst_i, gsems[0]),
    ]

    zero = jnp.zeros((16,), jnp.float32)

    def zbuf_body(i, carry):
        rows[0][i // 8, pl.ds((i % 8) * 16, 16)] = zero
        return carry

    lax.fori_loop(0, CHUNK * (D // 16), zbuf_body, 0)

    n_z = (NZC - s + NS - 1) // NS

    def zacc_body(t, carry):
        j = s + t * NS
        pltpu.sync_copy(rows[0], acc.at[pl.ds(j * CHUNK, CHUNK)])
        return carry

    lax.fori_loop(0, n_z, zacc_body, 0)
    plsc.subcore_barrier()
    for p in pref:
        p.wait()

    def fire_gather(j, b):
        pltpu.async_copy(x_hbm.at[src_i.at[j]], rows[b], gsems[b])

    def wait_gather(b):
        pltpu.make_async_copy(x_hbm.at[src_i.at[0]], rows[b],
                              gsems[b]).wait()

    def fire_scatter(j, b):
        pltpu.async_copy(rows[b], acc.at[dst_i.at[j]], ssems[b], add=True)

    def wait_scatter(b):
        pltpu.make_async_copy(rows[b], acc.at[dst_i.at[0]],
                              ssems[b]).wait()

    def pipe_body(t, carry):
        j0 = t * NB
        for b in range(NB):
            @pl.when(j0 + b < CPH)
            def _():
                wait_gather(b)
                fire_scatter(j0 + b, b)
        for b in range(NB):
            @pl.when(j0 + b + NB < CPH)
            def _():
                wait_scatter(b)
                fire_gather(j0 + b + NB, b)
        return carry

    for h in range(NHALF):
        if h > 0:
            pltpu.sync_copy(src_hbm.at[wid, h], src_i)
            pltpu.sync_copy(dst_hbm.at[wid, h], dst_i)
        for b in range(NB):
            fire_gather(b, b)
        lax.fori_loop(0, -(-CPH // NB), pipe_body, 0)
        for b in range(NB):
            wait_scatter(b)
    plsc.subcore_barrier()

    n_wb = (NZC - s + NS - 1) // NS

    def wb_body(t, carry):
        j = s + t * NS
        pltpu.sync_copy(acc.at[pl.ds(j * CHUNK, CHUNK)],
                        out_hbm.at[c, pl.ds(j * CHUNK, CHUNK)])
        return carry

    lax.fori_loop(0, n_wb, wb_body, 0)


def _sc_aggregate(edge_index, x):
    src3 = edge_index[0].reshape(NW, NHALF, CPH, CHUNK)
    dst3 = edge_index[1].reshape(NW, NHALF, CPH, CHUNK)

    mesh = plsc.VectorSubcoreMesh(core_axis_name="c", subcore_axis_name="s")
    k = pl.kernel(
        _sc_aggregate_body,
        out_type=jax.ShapeDtypeStruct((NC, N_NODES, D), jnp.float32),
        mesh=mesh,
        scratch_types=[
            pltpu.VMEM((CPH, CHUNK), jnp.int32),
            pltpu.VMEM((CPH, CHUNK), jnp.int32),
            [pltpu.VMEM((CHUNK, D), jnp.float32) for _ in range(NB)],
            pltpu.VMEM_SHARED((N_NODES, D), jnp.float32),
            [pltpu.SemaphoreType.DMA for _ in range(NB)],
            [pltpu.SemaphoreType.DMA for _ in range(NB)],
        ],
    )
    return k(src3, dst3, x)


def _tc_head_body(x_ref, p_ref, ws_ref, wn_ref, b_ref, wo_ref, out_ref):
    agg = p_ref[0] + p_ref[1]
    h = (jnp.dot(x_ref[...], ws_ref[...], preferred_element_type=jnp.float32)
         + jnp.dot(agg, wn_ref[...], preferred_element_type=jnp.float32)
         + b_ref[...][None, :])
    h = jnp.maximum(h, 0.0)
    out_ref[...] = jnp.sum(h * wo_ref[...][None, :], axis=1)


def _tc_head(x, partials, W_self, W_nbr, b, w_out):
    return pl.pallas_call(
        _tc_head_body,
        out_shape=jax.ShapeDtypeStruct((N_NODES,), jnp.float32),
    )(x, partials, W_self, W_nbr, b, w_out)


def kernel(x, edge_index, W_self, W_nbr, b, w_out):
    partials = _sc_aggregate(edge_index, x)
    return _tc_head(x, partials, W_self, W_nbr, b, w_out)
